# Initial kernel scaffold; baseline (speedup 1.0000x reference)
import jax
import jax.numpy as jnp
from jax.experimental import pallas as pl


def kernel(x, edge_index, W1, b1, W2, b2, W3, b3):
    raise NotImplementedError("write your pallas kernel here")


# speedup vs baseline: 33.5407x; 33.5407x over previous
import functools

import jax
import jax.numpy as jnp
from jax import lax
from jax.experimental import pallas as pl
from jax.experimental.pallas import tpu as pltpu
from jax.experimental.pallas import tpu_sc as plsc

_N = 10000
_NACC = 10240
_RPT = 640
_STEP = 128
_NSTEP = 80
_NW = 32
_EPAD = _NW * _NSTEP * _STEP


def _sc_mesh():
    return plsc.VectorSubcoreMesh(core_axis_name="c", subcore_axis_name="s")


def _make_scatter(F):
    two_d = F > 1
    gshape = (_NACC, F) if two_d else (_NACC,)
    bufshape = (_STEP, F) if two_d else (_STEP,)
    stshape = (_RPT, F) if two_d else (_RPT,)

    @functools.partial(
        pl.kernel,
        out_type=jax.ShapeDtypeStruct((2,) + gshape, jnp.float32),
        mesh=_sc_mesh(),
        compiler_params=pltpu.CompilerParams(use_tc_tiling_on_sc=False),
        scratch_types=[
            pltpu.VMEM((_NSTEP, _STEP), jnp.int32),
            pltpu.VMEM((_NSTEP, _STEP), jnp.int32),
            pltpu.VMEM(bufshape, jnp.float32),
            pltpu.VMEM(bufshape, jnp.float32),
            pltpu.VMEM(stshape, jnp.float32),
            pltpu.VMEM_SHARED(gshape, jnp.float32),
            pltpu.SemaphoreType.DMA,
            pltpu.SemaphoreType.DMA,
        ],
    )
    def scat(g_hbm, row_hbm, col_hbm, out_hbm,
             row_v, col_v, buf0, buf1, stage, acc_sh, sem0, sem1):
        c = lax.axis_index("c")
        s = lax.axis_index("s")
        wid = s * 2 + c
        rbase = s * _RPT

        if two_d:
            pltpu.sync_copy(g_hbm.at[pl.ds(rbase, _RPT), :], stage)
            pltpu.sync_copy(stage, acc_sh.at[pl.ds(rbase, _RPT), :])
        else:
            pltpu.sync_copy(g_hbm.at[pl.ds(rbase, _RPT)], stage)
            pltpu.sync_copy(stage, acc_sh.at[pl.ds(rbase, _RPT)])
        pltpu.sync_copy(row_hbm.at[wid], row_v)
        pltpu.sync_copy(col_hbm.at[wid], col_v)
        plsc.subcore_barrier()

        def gat(j, buf, sem):
            return pltpu.make_async_copy(g_hbm.at[row_v.at[j]], buf, sem)

        gat(0, buf0, sem0).start()

        def body(k, carry):
            j0 = 2 * k
            j1 = j0 + 1
            gat(j0, buf0, sem0).wait()
            gat(j1, buf1, sem1).start()
            pltpu.sync_copy(buf0, acc_sh.at[col_v.at[j0]], add=True)
            gat(j1, buf1, sem1).wait()

            @pl.when(k + 1 < _NSTEP // 2)
            def _():
                gat(j0 + 2, buf0, sem0).start()

            pltpu.sync_copy(buf1, acc_sh.at[col_v.at[j1]], add=True)
            return carry

        lax.fori_loop(0, _NSTEP // 2, body, 0)
        plsc.subcore_barrier()

        if two_d:
            pltpu.sync_copy(acc_sh.at[pl.ds(rbase, _RPT), :], stage)
            pltpu.sync_copy(stage, out_hbm.at[c, pl.ds(rbase, _RPT), :])
        else:
            pltpu.sync_copy(acc_sh.at[pl.ds(rbase, _RPT)], stage)
            pltpu.sync_copy(stage, out_hbm.at[c, pl.ds(rbase, _RPT)])

    return scat


_scat64 = _make_scatter(64)
_scat32 = _make_scatter(32)
_scat1 = _make_scatter(1)


@functools.partial(
    pl.kernel,
    out_type=jax.ShapeDtypeStruct((2, _NACC), jnp.float32),
    mesh=_sc_mesh(),
    scratch_types=[
        pltpu.VMEM((_NSTEP, _STEP), jnp.int32),
        pltpu.VMEM((_STEP,), jnp.float32),
        pltpu.VMEM((_RPT,), jnp.float32),
        pltpu.VMEM_SHARED((_NACC,), jnp.float32),
    ],
)
def _deg_kernel(col_hbm, out_hbm, col_v, ones_v, stage, acc_sh):
    c = lax.axis_index("c")
    s = lax.axis_index("s")
    wid = s * 2 + c
    rbase = s * _RPT

    pltpu.sync_copy(col_hbm.at[wid], col_v)
    for i in range(_STEP // 16):
        ones_v[pl.ds(i * 16, 16)] = jnp.ones((16,), jnp.float32)
    for i in range(_RPT // 16):
        stage[pl.ds(i * 16, 16)] = jnp.zeros((16,), jnp.float32)
    pltpu.sync_copy(stage, acc_sh.at[pl.ds(rbase, _RPT)])
    plsc.subcore_barrier()

    def body(j, carry):
        pltpu.sync_copy(ones_v, acc_sh.at[col_v.at[j]], add=True)
        return carry

    lax.fori_loop(0, _NSTEP, body, 0)
    plsc.subcore_barrier()

    pltpu.sync_copy(acc_sh.at[pl.ds(rbase, _RPT)], stage)
    pltpu.sync_copy(stage, out_hbm.at[c, pl.ds(rbase, _RPT)])


def _tc1(dp3, x, W1):
    def body(dp_ref, x_ref, w_ref, g_ref, dinv_ref):
        dp = dp_ref[...]
        deg = dp[0, :_N] + dp[1, :_N] + 1.0
        dinv = lax.rsqrt(deg)
        h = jnp.dot(x_ref[...], w_ref[...], preferred_element_type=jnp.float32)
        g_ref[pl.ds(0, _N), :] = dinv * h
        g_ref[pl.ds(_N, _NACC - _N), :] = jnp.zeros((_NACC - _N, w_ref.shape[1]),
                                                    jnp.float32)
        dinv_ref[...] = dinv

    return pl.pallas_call(
        body,
        out_shape=[
            jax.ShapeDtypeStruct((_NACC, W1.shape[1]), jnp.float32),
            jax.ShapeDtypeStruct((_N, 1), jnp.float32),
        ],
    )(dp3, x, W1)


def _tc_mid(sp, g, dinv, b_row, W_next):
    def body(sp_ref, g_ref, dinv_ref, b_ref, w_ref, o_ref):
        sp_ = sp_ref[...]
        g_ = g_ref[pl.ds(0, _N), :]
        dinv = dinv_ref[...]
        h = sp_[0, :_N] + sp_[1, :_N] - g_
        h = jnp.maximum(dinv * h + b_ref[...], 0.0)
        g_next = dinv * jnp.dot(h, w_ref[...], preferred_element_type=jnp.float32)
        o_ref[pl.ds(0, _N), :] = g_next
        o_ref[pl.ds(_N, _NACC - _N), :] = jnp.zeros((_NACC - _N, w_ref.shape[1]),
                                                    jnp.float32)

    return pl.pallas_call(
        body,
        out_shape=jax.ShapeDtypeStruct((_NACC, W_next.shape[1]), jnp.float32),
    )(sp, g, dinv, b_row, W_next)


def _tc3(sp, g, dinv, b_row, w3_row):
    def body(sp_ref, g_ref, dinv_ref, b_ref, w_ref, o_ref):
        sp_ = sp_ref[...]
        g_ = g_ref[pl.ds(0, _N), :]
        dinv = dinv_ref[...]
        h = sp_[0, :_N] + sp_[1, :_N] - g_
        h = jnp.maximum(dinv * h + b_ref[...], 0.0)
        hw = jnp.sum(h * w_ref[...], axis=1, keepdims=True)
        o_ref[pl.ds(0, _N), :] = dinv * hw
        o_ref[pl.ds(_N, _NACC - _N), :] = jnp.zeros((_NACC - _N, 1), jnp.float32)

    return pl.pallas_call(
        body,
        out_shape=jax.ShapeDtypeStruct((_NACC, 1), jnp.float32),
    )(sp, g, dinv, b_row, w3_row)


def _tc4(sp3, g3, dinv, b3_sc):
    def body(sp_ref, g_ref, dinv_ref, b_ref, o_ref):
        sp_ = sp_ref[...]
        s = sp_[0, :_N] + sp_[1, :_N] - g_ref[pl.ds(0, _N), :]
        o_ref[...] = dinv_ref[...] * s + b_ref[...]

    return pl.pallas_call(
        body,
        out_shape=jax.ShapeDtypeStruct((_N, 1), jnp.float32),
    )(sp3, g3, dinv, b3_sc)


def kernel(x, edge_index, W1, b1, W2, b2, W3, b3):
    row = edge_index[0].astype(jnp.int32)
    col = edge_index[1].astype(jnp.int32)
    e = row.shape[0]
    pad = (jnp.arange(_EPAD - e, dtype=jnp.int32) % (_NACC - _N)) + _N
    rowp = jnp.concatenate([row, pad]).reshape(_NW, _NSTEP, _STEP)
    colp = jnp.concatenate([col, pad]).reshape(_NW, _NSTEP, _STEP)

    degp = _deg_kernel(colp)
    g1, dinv = _tc1(degp.reshape(2, _NACC, 1), x, W1)
    s1 = _scat64(g1, rowp, colp)
    g2 = _tc_mid(s1, g1, dinv, b1.reshape(1, -1), W2)
    s2 = _scat32(g2, rowp, colp)
    g3 = _tc3(s2, g2, dinv, b2.reshape(1, -1), W3.reshape(1, -1))
    s3 = _scat1(g3.reshape(_NACC), rowp, colp)
    out = _tc4(s3.reshape(2, _NACC, 1), g3, dinv, b3.reshape(1, 1))
    return out.reshape(-1)


# speedup vs baseline: 46.9061x; 1.3985x over previous
import functools

import jax
import jax.numpy as jnp
from jax import lax
from jax.experimental import pallas as pl
from jax.experimental.pallas import tpu as pltpu
from jax.experimental.pallas import tpu_sc as plsc

_N = 10000
_NACC = 10240
_RPT = 640
_STEP = 128
_NSTEP = 80
_NW = 32
_NBUF = 4
_EPAD = _NW * _NSTEP * _STEP


def _sc_mesh():
    return plsc.VectorSubcoreMesh(core_axis_name="c", subcore_axis_name="s")


def _make_scatter(F):
    two_d = F > 1
    gshape = (_NACC, F) if two_d else (_NACC,)
    bufshape = (_STEP, F) if two_d else (_STEP,)
    stshape = (_RPT, F) if two_d else (_RPT,)

    @functools.partial(
        pl.kernel,
        out_type=jax.ShapeDtypeStruct((2,) + gshape, jnp.float32),
        mesh=_sc_mesh(),
        compiler_params=pltpu.CompilerParams(use_tc_tiling_on_sc=False),
        scratch_types=[
            pltpu.VMEM((_NSTEP, _STEP), jnp.int32),
            pltpu.VMEM((_NSTEP, _STEP), jnp.int32),
            [pltpu.VMEM(bufshape, jnp.float32) for _ in range(_NBUF)],
            pltpu.VMEM_SHARED(gshape, jnp.float32),
            [pltpu.SemaphoreType.DMA for _ in range(_NBUF)],
        ],
    )
    def scat(g_hbm, row_hbm, col_hbm, out_hbm,
             row_v, col_v, bufs, acc_sh, sems):
        c = lax.axis_index("c")
        s = lax.axis_index("s")
        wid = s * 2 + c
        rbase = s * _RPT

        if two_d:
            pltpu.sync_copy(g_hbm.at[pl.ds(rbase, _RPT), :],
                            acc_sh.at[pl.ds(rbase, _RPT), :])
        else:
            pltpu.sync_copy(g_hbm.at[pl.ds(rbase, _RPT)],
                            acc_sh.at[pl.ds(rbase, _RPT)])
        pltpu.sync_copy(row_hbm.at[wid], row_v)
        pltpu.sync_copy(col_hbm.at[wid], col_v)
        plsc.subcore_barrier()

        def gat(j, b):
            return pltpu.make_async_copy(g_hbm.at[row_v.at[j]], bufs[b], sems[b])

        for b in range(_NBUF):
            gat(b, b).start()

        def body(k, carry):
            for b in range(_NBUF):
                j = _NBUF * k + b
                gat(j, b).wait()
                pltpu.sync_copy(bufs[b], acc_sh.at[col_v.at[j]], add=True)

                @pl.when(j + _NBUF < _NSTEP)
                def _():
                    gat(j + _NBUF, b).start()

            return carry

        lax.fori_loop(0, _NSTEP // _NBUF, body, 0)
        plsc.subcore_barrier()

        if two_d:
            pltpu.sync_copy(acc_sh.at[pl.ds(rbase, _RPT), :],
                            out_hbm.at[c, pl.ds(rbase, _RPT), :])
        else:
            pltpu.sync_copy(acc_sh.at[pl.ds(rbase, _RPT)],
                            out_hbm.at[c, pl.ds(rbase, _RPT)])

    return scat


_scat64 = _make_scatter(64)
_scat32 = _make_scatter(32)
_scat1 = _make_scatter(1)


@functools.partial(
    pl.kernel,
    out_type=jax.ShapeDtypeStruct((2, _NACC), jnp.float32),
    mesh=_sc_mesh(),
    scratch_types=[
        pltpu.VMEM((_NSTEP, _STEP), jnp.int32),
        pltpu.VMEM((_STEP,), jnp.float32),
        pltpu.VMEM((_RPT,), jnp.float32),
        pltpu.VMEM_SHARED((_NACC,), jnp.float32),
    ],
)
def _deg_kernel(col_hbm, out_hbm, col_v, ones_v, stage, acc_sh):
    c = lax.axis_index("c")
    s = lax.axis_index("s")
    wid = s * 2 + c
    rbase = s * _RPT

    pltpu.sync_copy(col_hbm.at[wid], col_v)
    for i in range(_STEP // 16):
        ones_v[pl.ds(i * 16, 16)] = jnp.ones((16,), jnp.float32)
    for i in range(_RPT // 16):
        stage[pl.ds(i * 16, 16)] = jnp.zeros((16,), jnp.float32)
    pltpu.sync_copy(stage, acc_sh.at[pl.ds(rbase, _RPT)])
    plsc.subcore_barrier()

    def body(j, carry):
        pltpu.sync_copy(ones_v, acc_sh.at[col_v.at[j]], add=True)
        return carry

    lax.fori_loop(0, _NSTEP, body, 0)
    plsc.subcore_barrier()

    pltpu.sync_copy(acc_sh.at[pl.ds(rbase, _RPT)], stage)
    pltpu.sync_copy(stage, out_hbm.at[c, pl.ds(rbase, _RPT)])


def _tc1(dp3, x, W1):
    def body(dp_ref, x_ref, w_ref, g_ref, dinv_ref):
        dp = dp_ref[...]
        deg = dp[0, :_N] + dp[1, :_N] + 1.0
        dinv = 1.0 / jnp.sqrt(deg)
        h = x_ref[...] @ w_ref[...]
        g_ref[pl.ds(0, _N), :] = dinv * h
        g_ref[pl.ds(_N, _NACC - _N), :] = jnp.zeros((_NACC - _N, w_ref.shape[1]),
                                                    jnp.float32)
        dinv_ref[...] = dinv

    return pl.pallas_call(
        body,
        out_shape=[
            jax.ShapeDtypeStruct((_NACC, W1.shape[1]), jnp.float32),
            jax.ShapeDtypeStruct((_N, 1), jnp.float32),
        ],
    )(dp3, x, W1)


def _tc_mid(sp, g, dinv, b_row, W_next):
    def body(sp_ref, g_ref, dinv_ref, b_ref, w_ref, o_ref):
        sp_ = sp_ref[...]
        g_ = g_ref[pl.ds(0, _N), :]
        dinv = dinv_ref[...]
        h = sp_[0, :_N] + sp_[1, :_N] - g_
        h = jnp.maximum(dinv * h + b_ref[...], 0.0)
        g_next = dinv * (h @ w_ref[...])
        o_ref[pl.ds(0, _N), :] = g_next
        o_ref[pl.ds(_N, _NACC - _N), :] = jnp.zeros((_NACC - _N, w_ref.shape[1]),
                                                    jnp.float32)

    return pl.pallas_call(
        body,
        out_shape=jax.ShapeDtypeStruct((_NACC, W_next.shape[1]), jnp.float32),
    )(sp, g, dinv, b_row, W_next)


def _tc3(sp, g, dinv, b_row, w3_col):
    def body(sp_ref, g_ref, dinv_ref, b_ref, w_ref, o_ref):
        sp_ = sp_ref[...]
        g_ = g_ref[pl.ds(0, _N), :]
        dinv = dinv_ref[...]
        h = sp_[0, :_N] + sp_[1, :_N] - g_
        h = jnp.maximum(dinv * h + b_ref[...], 0.0)
        hw = h @ w_ref[...]
        o_ref[pl.ds(0, _N), :] = dinv * hw
        o_ref[pl.ds(_N, _NACC - _N), :] = jnp.zeros((_NACC - _N, 1), jnp.float32)

    return pl.pallas_call(
        body,
        out_shape=jax.ShapeDtypeStruct((_NACC, 1), jnp.float32),
    )(sp, g, dinv, b_row, w3_col)


def _tc4(sp3, g3, dinv, b3_sc):
    def body(sp_ref, g_ref, dinv_ref, b_ref, o_ref):
        sp_ = sp_ref[...]
        s = sp_[0, :_N] + sp_[1, :_N] - g_ref[pl.ds(0, _N), :]
        o_ref[...] = dinv_ref[...] * s + b_ref[...]

    return pl.pallas_call(
        body,
        out_shape=jax.ShapeDtypeStruct((_N, 1), jnp.float32),
    )(sp3, g3, dinv, b3_sc)


def kernel(x, edge_index, W1, b1, W2, b2, W3, b3):
    row = edge_index[0].astype(jnp.int32)
    col = edge_index[1].astype(jnp.int32)
    e = row.shape[0]
    pad = (jnp.arange(_EPAD - e, dtype=jnp.int32) % (_NACC - _N)) + _N
    rowp = jnp.concatenate([row, pad]).reshape(_NW, _NSTEP, _STEP)
    colp = jnp.concatenate([col, pad]).reshape(_NW, _NSTEP, _STEP)

    degp = _deg_kernel(colp)
    g1, dinv = _tc1(degp.reshape(2, _NACC, 1), x, W1)
    s1 = _scat64(g1, rowp, colp)
    g2 = _tc_mid(s1, g1, dinv, b1.reshape(1, -1), W2)
    s2 = _scat32(g2, rowp, colp)
    g3 = _tc3(s2, g2, dinv, b2.reshape(1, -1), W3)
    s3 = _scat1(g3.reshape(_NACC), rowp, colp)
    out = _tc4(s3.reshape(2, _NACC, 1), g3, dinv, b3.reshape(1, 1))
    return out.reshape(-1)


# speedup vs baseline: 46.9857x; 1.0017x over previous
import functools

import jax
import jax.numpy as jnp
from jax import lax
from jax.experimental import pallas as pl
from jax.experimental.pallas import tpu as pltpu
from jax.experimental.pallas import tpu_sc as plsc

_N = 10000
_NACC = 10240
_RPT = 640
_STEP = 128
_NSTEP = 80
_NW = 32
_NBUF = 4
_EPAD = _NW * _NSTEP * _STEP


def _sc_mesh():
    return plsc.VectorSubcoreMesh(core_axis_name="c", subcore_axis_name="s")


def _make_scatter(F):
    two_d = F > 1
    gshape = (_NACC, F) if two_d else (_NACC,)
    bufshape = (_STEP, F) if two_d else (_STEP,)

    @functools.partial(
        pl.kernel,
        out_type=jax.ShapeDtypeStruct((2,) + gshape, jnp.float32),
        mesh=_sc_mesh(),
        compiler_params=pltpu.CompilerParams(use_tc_tiling_on_sc=False),
        scratch_types=[
            pltpu.VMEM((_NSTEP, _STEP), jnp.int32),
            pltpu.VMEM((_NSTEP, _STEP), jnp.int32),
            [pltpu.VMEM(bufshape, jnp.float32) for _ in range(_NBUF)],
            pltpu.VMEM_SHARED(gshape, jnp.float32),
            [pltpu.SemaphoreType.DMA for _ in range(_NBUF)],
        ],
    )
    def scat(g_hbm, row_hbm, col_hbm, out_hbm,
             row_v, col_v, bufs, acc_sh, sems):
        c = lax.axis_index("c")
        s = lax.axis_index("s")
        wid = s * 2 + c
        rbase = s * _RPT

        if two_d:
            pltpu.sync_copy(g_hbm.at[pl.ds(rbase, _RPT), :],
                            acc_sh.at[pl.ds(rbase, _RPT), :])
        else:
            pltpu.sync_copy(g_hbm.at[pl.ds(rbase, _RPT)],
                            acc_sh.at[pl.ds(rbase, _RPT)])
        pltpu.sync_copy(row_hbm.at[wid], row_v)
        pltpu.sync_copy(col_hbm.at[wid], col_v)
        plsc.subcore_barrier()

        def gat(j, b):
            return pltpu.make_async_copy(g_hbm.at[row_v.at[j]], bufs[b], sems[b])

        for b in range(_NBUF):
            gat(b, b).start()

        def body(k, carry):
            for b in range(_NBUF):
                j = _NBUF * k + b
                gat(j, b).wait()
                pltpu.sync_copy(bufs[b], acc_sh.at[col_v.at[j]], add=True)

                @pl.when(j + _NBUF < _NSTEP)
                def _():
                    gat(j + _NBUF, b).start()

            return carry

        lax.fori_loop(0, _NSTEP // _NBUF, body, 0)
        plsc.subcore_barrier()

        if two_d:
            pltpu.sync_copy(acc_sh.at[pl.ds(rbase, _RPT), :],
                            out_hbm.at[c, pl.ds(rbase, _RPT), :])
        else:
            pltpu.sync_copy(acc_sh.at[pl.ds(rbase, _RPT)],
                            out_hbm.at[c, pl.ds(rbase, _RPT)])

    return scat


_scat64 = _make_scatter(64)
_scat32 = _make_scatter(32)
_scat1 = _make_scatter(1)


@functools.partial(
    pl.kernel,
    out_type=jax.ShapeDtypeStruct((2, _NACC), jnp.float32),
    mesh=_sc_mesh(),
    scratch_types=[
        pltpu.VMEM((_NSTEP, _STEP), jnp.int32),
        pltpu.VMEM((_STEP,), jnp.float32),
        pltpu.VMEM((_RPT,), jnp.float32),
        pltpu.VMEM_SHARED((_NACC,), jnp.float32),
    ],
)
def _deg_kernel(col_hbm, out_hbm, col_v, ones_v, stage, acc_sh):
    c = lax.axis_index("c")
    s = lax.axis_index("s")
    wid = s * 2 + c
    rbase = s * _RPT

    pltpu.sync_copy(col_hbm.at[wid], col_v)
    for i in range(_STEP // 16):
        ones_v[pl.ds(i * 16, 16)] = jnp.ones((16,), jnp.float32)
    for i in range(_RPT // 16):
        stage[pl.ds(i * 16, 16)] = jnp.zeros((16,), jnp.float32)
    pltpu.sync_copy(stage, acc_sh.at[pl.ds(rbase, _RPT)])
    plsc.subcore_barrier()

    def body(j, carry):
        pltpu.sync_copy(ones_v, acc_sh.at[col_v.at[j]], add=True)
        return carry

    lax.fori_loop(0, _NSTEP, body, 0)
    plsc.subcore_barrier()

    pltpu.sync_copy(acc_sh.at[pl.ds(rbase, _RPT)], stage)
    pltpu.sync_copy(stage, out_hbm.at[c, pl.ds(rbase, _RPT)])


def _tc1(dp3, x, W1):
    def body(dp_ref, x_ref, w_ref, g_ref, dinv_ref):
        dp = dp_ref[...]
        deg = dp[0, :_N] + dp[1, :_N] + 1.0
        dinv = 1.0 / jnp.sqrt(deg)
        h = x_ref[...] @ w_ref[...]
        g_ref[pl.ds(0, _N), :] = dinv * h
        g_ref[pl.ds(_N, _NACC - _N), :] = jnp.zeros((_NACC - _N, w_ref.shape[1]),
                                                    jnp.float32)
        dinv_ref[...] = dinv

    return pl.pallas_call(
        body,
        out_shape=[
            jax.ShapeDtypeStruct((_NACC, W1.shape[1]), jnp.float32),
            jax.ShapeDtypeStruct((_N, 1), jnp.float32),
        ],
    )(dp3, x, W1)


def _tc_mid(sp, g, dinv, b_row, W_next):
    def body(sp_ref, g_ref, dinv_ref, b_ref, w_ref, o_ref):
        sp_ = sp_ref[...]
        g_ = g_ref[pl.ds(0, _N), :]
        dinv = dinv_ref[...]
        h = sp_[0, :_N] + sp_[1, :_N] - g_
        h = jnp.maximum(dinv * h + b_ref[...], 0.0)
        g_next = dinv * (h @ w_ref[...])
        o_ref[pl.ds(0, _N), :] = g_next
        o_ref[pl.ds(_N, _NACC - _N), :] = jnp.zeros((_NACC - _N, w_ref.shape[1]),
                                                    jnp.float32)

    return pl.pallas_call(
        body,
        out_shape=jax.ShapeDtypeStruct((_NACC, W_next.shape[1]), jnp.float32),
    )(sp, g, dinv, b_row, W_next)


def _tc3(sp, g, dinv, b_row, w3_col):
    def body(sp_ref, g_ref, dinv_ref, b_ref, w_ref, o_ref):
        sp_ = sp_ref[...]
        g_ = g_ref[pl.ds(0, _N), :]
        dinv = dinv_ref[...]
        h = sp_[0, :_N] + sp_[1, :_N] - g_
        h = jnp.maximum(dinv * h + b_ref[...], 0.0)
        hw = h @ w_ref[...]
        o_ref[pl.ds(0, _N), :] = dinv * hw
        o_ref[pl.ds(_N, _NACC - _N), :] = jnp.zeros((_NACC - _N, 1), jnp.float32)

    return pl.pallas_call(
        body,
        out_shape=jax.ShapeDtypeStruct((_NACC, 1), jnp.float32),
    )(sp, g, dinv, b_row, w3_col)


def _tc4(sp3, g3, dinv, b3_sc):
    def body(sp_ref, g_ref, dinv_ref, b_ref, o_ref):
        sp_ = sp_ref[...]
        s = sp_[0, :_N] + sp_[1, :_N] - g_ref[pl.ds(0, _N), :]
        o_ref[...] = dinv_ref[...] * s + b_ref[...]

    return pl.pallas_call(
        body,
        out_shape=jax.ShapeDtypeStruct((_N, 1), jnp.float32),
    )(sp3, g3, dinv, b3_sc)


def kernel(x, edge_index, W1, b1, W2, b2, W3, b3):
    row = edge_index[0].astype(jnp.int32)
    col = edge_index[1].astype(jnp.int32)
    e = row.shape[0]
    pad = (jnp.arange(_EPAD - e, dtype=jnp.int32) % (_NACC - _N)) + _N
    rowp = jnp.concatenate([row, pad]).reshape(_NW, _NSTEP, _STEP)
    colp = jnp.concatenate([col, pad]).reshape(_NW, _NSTEP, _STEP)

    degp = _deg_kernel(colp)
    g1, dinv = _tc1(degp.reshape(2, _NACC, 1), x, W1)
    s1 = _scat64(g1, rowp, colp)
    g2 = _tc_mid(s1, g1, dinv, b1.reshape(1, -1), W2)
    s2 = _scat32(g2, rowp, colp)
    g3 = _tc3(s2, g2, dinv, b2.reshape(1, -1), W3)
    s3 = _scat1(g3.reshape(_NACC), rowp, colp)
    out = _tc4(s3.reshape(2, _NACC, 1), g3, dinv, b3.reshape(1, 1))
    return out.reshape(-1)
